# baseline (speedup 1.0000x reference)
import functools

import jax
import jax.numpy as jnp
import numpy as np
from jax import lax
from jax.experimental import pallas as pl
from jax.experimental.pallas import tpu as pltpu
from jax.experimental.pallas import tpu_sc as plsc

N = 32768
TOP_K = 25
NS = 16
CHUNK = N // NS
NSEG = 8
SEG = CHUNK // NSEG
SEGV = SEG // 16
CAND = 32
ILP = 4
NEG_INF = np.float32(-np.inf)
BIG_I32 = np.int32(2**31 - 1)

_mesh = plsc.VectorSubcoreMesh(core_axis_name="c", subcore_axis_name="s",
                               num_cores=1)


@functools.partial(
    pl.kernel,
    mesh=_mesh,
    out_type=jax.ShapeDtypeStruct((N,), jnp.float32),
    compiler_params=pltpu.CompilerParams(needs_layout_passes=False),
    scratch_types=[
        pltpu.VMEM((CHUNK,), jnp.float32),
        pltpu.VMEM((CHUNK,), jnp.float32),
        pltpu.VMEM((2 * CAND,), jnp.int32),
        pltpu.VMEM((NS * 2 * CAND,), jnp.int32),
        pltpu.VMEM((CAND,), jnp.float32),
        pltpu.VMEM((CAND,), jnp.int32),
        pltpu.VMEM_SHARED((NS * 2 * CAND,), jnp.int32),
        pltpu.VMEM_SHARED((NS * 16,), jnp.int32),
        pltpu.SemaphoreType.DMA,
        pltpu.SemaphoreType.DMA,
    ],
)
def _topk_mask_kernel(x_hbm, out_hbm, w, zbuf, cb, mb,
                      sv, si, cb_sh, pool_sh, sem, zsem):
    s = lax.axis_index("s")
    lanes = lax.iota(jnp.int32, 16)
    lane0 = lanes == 0
    chunk_base = s * np.int32(CHUNK)

    icopy = pltpu.async_copy(x_hbm.at[pl.ds(s * CHUNK, CHUNK)], w, sem)

    zeros16 = jnp.zeros((16,), jnp.float32)

    def zfill(j, _):
        zbuf[pl.ds(j * 16, 16)] = zeros16
        return 0

    lax.fori_loop(0, CHUNK // 16, zfill, 0, unroll=4)
    zcopy = pltpu.async_copy(zbuf, out_hbm.at[pl.ds(s * CHUNK, CHUNK)], zsem)
    icopy.wait()

    neg = jnp.full((16,), NEG_INF, jnp.float32)
    zero = jnp.zeros((16,), jnp.int32)

    def better(a, b):
        av, ai = a
        bv, bi = b
        m = (bv > av) | ((bv == av) & (bi < ai))
        return jnp.where(m, bv, av), jnp.where(m, bi, ai)

    def scan_segment(base):
        def scan_body(q, carry):
            out = []
            for a in range(ILP):
                cmax, cpos = carry[a]
                j = q * ILP + a
                v = w[pl.ds(base + j * 16, 16)]
                m = v > cmax
                out.append((jnp.where(m, v, cmax),
                            jnp.where(m, jnp.full((16,), j, jnp.int32), cpos)))
            return tuple(out)

        acc = lax.fori_loop(0, SEGV // ILP, scan_body, ((neg, zero),) * ILP,
                            unroll=SEGV // ILP)

        def with_idx(a):
            cmax, cpos = a
            return cmax, cpos * 16 + lanes + base + chunk_base

        return functools.reduce(better, [with_idx(acc[a]) for a in range(ILP)])

    pool = [scan_segment(np.int32(g * SEG)) for g in range(NSEG)]

    red = functools.reduce(jnp.maximum, [p[0] for p in pool])
    si[pl.ds(0, 16)] = plsc.bitcast(red, jnp.int32)
    pltpu.sync_copy(si.at[pl.ds(0, 16)], pool_sh.at[pl.ds(s * 16, 16)])
    plsc.subcore_barrier()
    pltpu.sync_copy(pool_sh, mb.at[pl.ds(0, NS * 16)])
    m1 = neg
    m2 = neg
    for r in range(NS):
        v = plsc.bitcast(mb[pl.ds(r * 16, 16)], jnp.float32)
        gt1 = v > m1
        gt2 = v > m2
        m2 = jnp.where(gt1, m1, jnp.where(gt2, v, m2))
        m1 = jnp.where(gt1, v, m1)
    t_bound = jnp.min(m2)

    def lane_set(pair, i, val):
        a, b = pair
        return (jnp.where(lanes == i, val, a),
                jnp.where(lanes == i - 16, val, b))

    def ext_cond(carry):
        i, go, _, _, _ = carry
        return (i < TOP_K) & go

    def ext_body(carry):
        i, _, v01, i01, pool = carry
        red_v, red_i = functools.reduce(better, pool)
        gmax = jnp.max(red_v)
        gidx = jnp.min(jnp.where(red_v == gmax, red_i, BIG_I32))
        pos = gidx - chunk_base
        plsc.store_scatter(w, [jnp.full((16,), pos, jnp.int32)],
                           jnp.full((16,), NEG_INF, jnp.float32), mask=lane0)
        g = lax.shift_right_logical(pos, 8)
        fresh = scan_segment(g * np.int32(SEG))
        pool = tuple(
            (jnp.where(g == r, fresh[0], pool[r][0]),
             jnp.where(g == r, fresh[1], pool[r][1]))
            for r in range(NSEG))
        return (i + 1, gmax >= t_bound, lane_set(v01, i, gmax),
                lane_set(i01, i, gidx), pool)

    init_v = (neg, neg)
    init_i = (jnp.full((16,), -1, jnp.int32),) * 2
    _, _, (av0, av1), (ai0, ai1), _ = lax.while_loop(
        ext_cond, ext_body,
        (np.int32(0), np.True_, init_v, init_i, tuple(pool)))
    cb[pl.ds(0, 16)] = plsc.bitcast(av0, jnp.int32)
    cb[pl.ds(16, 16)] = plsc.bitcast(av1, jnp.int32)
    cb[pl.ds(CAND, 16)] = ai0
    cb[pl.ds(CAND + 16, 16)] = ai1

    pltpu.sync_copy(cb, cb_sh.at[pl.ds(s * (2 * CAND), 2 * CAND)])
    zcopy.wait()
    plsc.subcore_barrier()

    pltpu.sync_copy(cb_sh, mb)

    def rank_pass(u, carry):
        r0, r1 = carry

        def cnt_list(vk, ik):
            cnt = zero
            for sz in (16, 8, 4, 2, 1):
                p = u * (2 * CAND) + cnt + (sz - 1)
                lv = plsc.bitcast(plsc.load_gather(mb, [p]), jnp.float32)
                li = plsc.load_gather(mb, [p + CAND])
                gtr = (lv > vk) | ((lv == vk) & (li < ik))
                cnt = cnt + jnp.where(gtr, sz, 0).astype(jnp.int32)
            return cnt

        return r0 + cnt_list(av0, ai0), r1 + cnt_list(av1, ai1)

    r0, r1 = lax.fori_loop(0, NS, rank_pass, (zero, zero), unroll=4)
    keep0 = r0 < TOP_K
    keep1 = r1 < TOP_K

    atb = jnp.maximum(
        jnp.max(jnp.where(keep0 & (ai0 == chunk_base), av0, NEG_INF)),
        jnp.max(jnp.where(keep1 & (ai1 == chunk_base), av1, NEG_INF)))
    v_pad = jnp.where(atb == NEG_INF, np.float32(0.0), atb)

    sv[pl.ds(0, 16)] = jnp.where(keep0, av0, v_pad)
    sv[pl.ds(16, 16)] = jnp.where(keep1, av1, v_pad)
    si[pl.ds(0, 16)] = jnp.where(keep0, ai0, chunk_base)
    si[pl.ds(16, 16)] = jnp.where(keep1, ai1, chunk_base)

    pltpu.async_copy(sv, out_hbm.at[si], sem).wait()


def kernel(score_vector):
    out = _topk_mask_kernel(jnp.reshape(score_vector, (N,)))
    return jnp.reshape(out, (1, N))


# speedup vs baseline: 1.2662x; 1.2662x over previous
import functools

import jax
import jax.numpy as jnp
import numpy as np
from jax import lax
from jax.experimental import pallas as pl
from jax.experimental.pallas import tpu as pltpu
from jax.experimental.pallas import tpu_sc as plsc

N = 32768
TOP_K = 25
NS = 16
CHUNK = N // NS
NSEG = 8
SEG = CHUNK // NSEG
SEGV = SEG // 16
CAND = 32
ILP = 4
NEG_INF = np.float32(-np.inf)
BIG_I32 = np.int32(2**31 - 1)

_mesh = plsc.VectorSubcoreMesh(core_axis_name="c", subcore_axis_name="s",
                               num_cores=1)


@functools.partial(
    pl.kernel,
    mesh=_mesh,
    out_type=jax.ShapeDtypeStruct((N,), jnp.float32),
    compiler_params=pltpu.CompilerParams(needs_layout_passes=False),
    scratch_types=[
        pltpu.VMEM((CHUNK,), jnp.float32),
        pltpu.VMEM((CHUNK,), jnp.float32),
        pltpu.VMEM((2 * CAND,), jnp.int32),
        pltpu.VMEM((NS * 2 * CAND,), jnp.int32),
        pltpu.VMEM((CAND,), jnp.float32),
        pltpu.VMEM((CAND,), jnp.int32),
        pltpu.VMEM_SHARED((NS * 2 * CAND,), jnp.int32),
        pltpu.VMEM_SHARED((NS * 16,), jnp.int32),
        pltpu.SemaphoreType.DMA,
        pltpu.SemaphoreType.DMA,
    ],
)
def _topk_mask_kernel(x_hbm, out_hbm, w, zbuf, cb, mb,
                      sv, si, cb_sh, pool_sh, sem, zsem):
    s = lax.axis_index("s")
    lanes = lax.iota(jnp.int32, 16)
    lane0 = lanes == 0
    chunk_base = s * np.int32(CHUNK)

    icopy = pltpu.async_copy(x_hbm.at[pl.ds(s * CHUNK, CHUNK)], w, sem)

    zeros16 = jnp.zeros((16,), jnp.float32)

    def zfill(j, _):
        zbuf[pl.ds(j * 16, 16)] = zeros16
        return 0

    lax.fori_loop(0, CHUNK // 16, zfill, 0, unroll=4)
    zcopy = pltpu.async_copy(zbuf, out_hbm.at[pl.ds(s * CHUNK, CHUNK)], zsem)
    icopy.wait()

    neg = jnp.full((16,), NEG_INF, jnp.float32)
    zero = jnp.zeros((16,), jnp.int32)

    def better(a, b):
        av, ai = a
        bv, bi = b
        m = (bv > av) | ((bv == av) & (bi < ai))
        return jnp.where(m, bv, av), jnp.where(m, bi, ai)

    def scan_segment(base):
        def scan_body(q, carry):
            out = []
            for a in range(ILP):
                cmax, cpos = carry[a]
                j = q * ILP + a
                v = w[pl.ds(base + j * 16, 16)]
                m = v > cmax
                out.append((jnp.where(m, v, cmax),
                            jnp.where(m, jnp.full((16,), j, jnp.int32), cpos)))
            return tuple(out)

        acc = lax.fori_loop(0, SEGV // ILP, scan_body, ((neg, zero),) * ILP,
                            unroll=SEGV // ILP)

        def with_idx(a):
            cmax, cpos = a
            return cmax, cpos * 16 + lanes + base + chunk_base

        return functools.reduce(better, [with_idx(acc[a]) for a in range(ILP)])

    pool = [scan_segment(np.int32(g * SEG)) for g in range(NSEG)]

    red = functools.reduce(jnp.maximum, [p[0] for p in pool])
    si[pl.ds(0, 16)] = plsc.bitcast(red, jnp.int32)
    pltpu.sync_copy(si.at[pl.ds(0, 16)], pool_sh.at[pl.ds(s * 16, 16)])
    plsc.subcore_barrier()
    pltpu.sync_copy(pool_sh, mb.at[pl.ds(0, NS * 16)])
    m1 = neg
    m2 = neg
    for r in range(NS):
        v = plsc.bitcast(mb[pl.ds(r * 16, 16)], jnp.float32)
        gt1 = v > m1
        gt2 = v > m2
        m2 = jnp.where(gt1, m1, jnp.where(gt2, v, m2))
        m1 = jnp.where(gt1, v, m1)
    t_bound = jnp.min(m2)

    def lane_set(pair, i, val):
        a, b = pair
        return (jnp.where(lanes == i, val, a),
                jnp.where(lanes == i - 16, val, b))

    def ext_cond(carry):
        i, go, _, _, _ = carry
        return (i < TOP_K) & go

    def ext_body(carry):
        i, _, v01, i01, pool = carry
        red_v, red_i = functools.reduce(better, pool)
        gmax = jnp.max(red_v)
        gidx = jnp.min(jnp.where(red_v == gmax, red_i, BIG_I32))
        pos = gidx - chunk_base
        plsc.store_scatter(w, [jnp.full((16,), pos, jnp.int32)],
                           jnp.full((16,), NEG_INF, jnp.float32), mask=lane0)
        g = lax.shift_right_logical(pos, 8)
        fresh = scan_segment(g * np.int32(SEG))
        pool = tuple(
            (jnp.where(g == r, fresh[0], pool[r][0]),
             jnp.where(g == r, fresh[1], pool[r][1]))
            for r in range(NSEG))
        return (i + 1, gmax >= t_bound, lane_set(v01, i, gmax),
                lane_set(i01, i, gidx), pool)

    init_v = (neg, neg)
    init_i = (jnp.full((16,), -1, jnp.int32),) * 2
    _, _, (av0, av1), (ai0, ai1), _ = lax.while_loop(
        ext_cond, ext_body,
        (np.int32(0), np.True_, init_v, init_i, tuple(pool)))
    cb[pl.ds(0, 16)] = plsc.bitcast(av0, jnp.int32)
    cb[pl.ds(16, 16)] = plsc.bitcast(av1, jnp.int32)
    cb[pl.ds(CAND, 16)] = ai0
    cb[pl.ds(CAND + 16, 16)] = ai1

    pltpu.sync_copy(cb, cb_sh.at[pl.ds(s * (2 * CAND), 2 * CAND)])
    zcopy.wait()
    plsc.subcore_barrier()

    @pl.when(s == 0)
    def _merge_and_scatter():
        pltpu.sync_copy(cb_sh, mb)
        minus1 = jnp.full((16,), -1, jnp.int32)
        si[pl.ds(0, 16)] = minus1
        si[pl.ds(16, 16)] = minus1

        def merge(i, carry):
            ptr = carry
            pos = lanes * (2 * CAND) + ptr
            hv = plsc.bitcast(plsc.load_gather(mb, [pos]), jnp.float32)
            hi = plsc.load_gather(mb, [pos + CAND])
            won = hv == jnp.max(hv)
            wlane = lanes == plsc.all_reduce_ffs(won)
            isplat = jnp.full((16,), i, jnp.int32)
            plsc.store_scatter(sv, [isplat], hv, mask=wlane)
            plsc.store_scatter(si, [isplat], hi, mask=wlane)
            return ptr + jnp.where(wlane, 1, 0).astype(jnp.int32)

        lax.fori_loop(0, TOP_K, merge, zero)
        w0 = sv[pl.ds(0, 16)]
        w1 = sv[pl.ds(16, 16)]
        i0 = si[pl.ds(0, 16)]
        i1 = si[pl.ds(16, 16)]

        at0 = jnp.maximum(jnp.max(jnp.where(i0 == 0, w0, NEG_INF)),
                          jnp.max(jnp.where(i1 == 0, w1, NEG_INF)))
        v0 = jnp.where(at0 == NEG_INF, np.float32(0.0), at0)

        in0 = i0 >= 0
        in1 = i1 >= 0
        sv[pl.ds(0, 16)] = jnp.where(in0, w0, v0)
        sv[pl.ds(16, 16)] = jnp.where(in1, w1, v0)
        si[pl.ds(0, 16)] = jnp.where(in0, i0, 0)
        si[pl.ds(16, 16)] = jnp.where(in1, i1, 0)

        pltpu.async_copy(sv, out_hbm.at[si], sem).wait()


def kernel(score_vector):
    out = _topk_mask_kernel(jnp.reshape(score_vector, (N,)))
    return jnp.reshape(out, (1, N))
